# Initial kernel scaffold; baseline (speedup 1.0000x reference)
import jax
import jax.numpy as jnp
from jax.experimental import pallas as pl


def kernel(edge_index, uid, pos, neg, user_table, item_table, W1r, b1r, W1rb, b1rb, W2r, b2r, W2rb, b2rb):
    raise NotImplementedError("write your pallas kernel here")


# speedup vs baseline: 3.0259x; 3.0259x over previous
import functools

import jax
import jax.numpy as jnp
from jax import lax
from jax.experimental import pallas as pl
from jax.experimental.pallas import tpu as pltpu, tpu_sc as plsc

N = 10000
NPAD = 10240
D = 128
E = 320000
EPT = E // 16
K = 80
ITERS = EPT // K
RPT = NPAD // 16
B = 4096
NNEG = 4

_MESH = plsc.VectorSubcoreMesh(core_axis_name="c", subcore_axis_name="s")


def _sc_agg(feat0, feat1, src, dst, zf):
    @functools.partial(
        pl.kernel,
        out_type=jax.ShapeDtypeStruct((2 * NPAD, D), jnp.float32),
        mesh=_MESH,
        scratch_types=[pltpu.VMEM((K,), jnp.int32),
                       pltpu.VMEM((K,), jnp.int32),
                       pltpu.VMEM((K, D), jnp.float32),
                       pltpu.VMEM_SHARED((NPAD, D), jnp.float32),
                       pltpu.SemaphoreType.DMA])
    def k(feat0_h, feat1_h, src_h, dst_h, zf_h, acc_o,
          idxi, idxo, rows, acc_sh, sem):
        c = lax.axis_index("c")
        s = lax.axis_index("s")
        r0 = s * RPT
        pltpu.sync_copy(zf_h.at[pl.ds(r0, RPT)], acc_sh.at[pl.ds(r0, RPT)])
        plsc.subcore_barrier()

        def loop(feat_h, in_h, out_h):
            @pl.loop(0, ITERS)
            def chunk(i):
                base = s * EPT + i * K
                pltpu.sync_copy(in_h.at[pl.ds(base, K)], idxi)
                pltpu.sync_copy(out_h.at[pl.ds(base, K)], idxo)
                pltpu.async_copy(feat_h.at[idxi], rows, sem).wait()
                pltpu.sync_copy(rows, acc_sh.at[idxo], add=True)

        @pl.when(c == 0)
        def _():
            loop(feat0_h, src_h, dst_h)

        @pl.when(c == 1)
        def _():
            loop(feat1_h, dst_h, src_h)

        plsc.subcore_barrier()
        pltpu.sync_copy(acc_sh.at[pl.ds(r0, RPT)],
                        acc_o.at[pl.ds(c * NPAD + r0, RPT)])

    return k(feat0, feat1, src, dst, zf)


def _sc_deg(src, dst, zf, ones):
    @functools.partial(
        pl.kernel,
        out_type=jax.ShapeDtypeStruct((2 * NPAD, D), jnp.float32),
        mesh=_MESH,
        scratch_types=[pltpu.VMEM((K,), jnp.int32),
                       pltpu.VMEM((K, D), jnp.float32),
                       pltpu.VMEM_SHARED((NPAD, D), jnp.float32)])
    def k(src_h, dst_h, zf_h, ones_h, deg_o, idxo, ones_v, deg_sh):
        c = lax.axis_index("c")
        s = lax.axis_index("s")
        r0 = s * RPT
        pltpu.sync_copy(zf_h.at[pl.ds(r0, RPT)], deg_sh.at[pl.ds(r0, RPT)])
        pltpu.sync_copy(ones_h, ones_v)
        plsc.subcore_barrier()

        def loop(out_h):
            @pl.loop(0, ITERS)
            def chunk(i):
                base = s * EPT + i * K
                pltpu.sync_copy(out_h.at[pl.ds(base, K)], idxo)
                pltpu.sync_copy(ones_v, deg_sh.at[idxo], add=True)

        @pl.when(c == 0)
        def _():
            loop(dst_h)

        @pl.when(c == 1)
        def _():
            loop(src_h)

        plsc.subcore_barrier()
        pltpu.sync_copy(deg_sh.at[pl.ds(r0, RPT)],
                        deg_o.at[pl.ds(c * NPAD + r0, RPT)])

    return k(src, dst, zf, ones)


def _sc_gather(hu, hi, uid, posf, negf):
    @functools.partial(
        pl.kernel,
        out_type=[jax.ShapeDtypeStruct((B, D), jnp.float32),
                  jax.ShapeDtypeStruct((B, D), jnp.float32),
                  jax.ShapeDtypeStruct((B * NNEG, D), jnp.float32)],
        mesh=_MESH,
        scratch_types=[pltpu.VMEM((128,), jnp.int32),
                       pltpu.VMEM((128, D), jnp.float32),
                       pltpu.SemaphoreType.DMA])
    def k(hu_h, hi_h, uid_h, pos_h, neg_h, u_o, p_o, n_o, idx, rows, sem):
        c = lax.axis_index("c")
        s = lax.axis_index("s")
        wid = s * 2 + c

        def gath(tbl_h, ih, oh, base):
            pltpu.sync_copy(ih.at[pl.ds(base, 128)], idx)
            pltpu.async_copy(tbl_h.at[idx], rows, sem).wait()
            pltpu.sync_copy(rows, oh.at[pl.ds(base, 128)])

        gath(hu_h, uid_h, u_o, wid * 128)
        gath(hi_h, pos_h, p_o, wid * 128)
        for j in range(NNEG):
            gath(hi_h, neg_h, n_o, wid * 512 + j * 128)

    return k(hu, hi, uid, posf, negf)


def _tc_layer(acc, deg, Ws, bs, leaky):
    RB = 640

    def body(a_r, d_r, w_r, b_r, o_r):
        x = a_r[0]
        dcol = d_r[0][:, 0:1]
        m = x / jnp.maximum(dcol, 1.0)
        y = jnp.dot(m, w_r[0], preferred_element_type=jnp.float32)
        y = y + jnp.where(dcol > 0, b_r[0], 0.0)
        if leaky:
            y = jnp.where(y >= 0, y, 0.01 * y)
        o_r[0] = y

    return pl.pallas_call(
        body,
        grid=(2, NPAD // RB),
        in_specs=[pl.BlockSpec((1, RB, D), lambda d, i: (d, i, 0)),
                  pl.BlockSpec((1, RB, D), lambda d, i: (d, i, 0)),
                  pl.BlockSpec((1, D, D), lambda d, i: (d, 0, 0)),
                  pl.BlockSpec((1, 1, D), lambda d, i: (d, 0, 0))],
        out_specs=pl.BlockSpec((1, RB, D), lambda d, i: (d, i, 0)),
        out_shape=jax.ShapeDtypeStruct((2, NPAD, D), jnp.float32),
    )(acc, deg, Ws, bs)


def _tc_dots(U, Pi, N4):
    RB = 512

    def body(u_r, p_r, n_r, o_r):
        u = u_r[...]
        cols = [jnp.sum(u * p_r[...], axis=-1, keepdims=True)]
        for kk in range(NNEG):
            cols.append(jnp.sum(u * n_r[:, kk * D:(kk + 1) * D],
                                axis=-1, keepdims=True))
        cols.append(jnp.zeros((RB, 3), jnp.float32))
        o_r[...] = jnp.concatenate(cols, axis=-1)

    return pl.pallas_call(
        body,
        grid=(B // RB,),
        in_specs=[pl.BlockSpec((RB, D), lambda i: (i, 0)),
                  pl.BlockSpec((RB, D), lambda i: (i, 0)),
                  pl.BlockSpec((RB, NNEG * D), lambda i: (i, 0))],
        out_specs=pl.BlockSpec((RB, 8), lambda i: (i, 0)),
        out_shape=jax.ShapeDtypeStruct((B, 8), jnp.float32),
    )(U, Pi, N4)


def kernel(edge_index, uid, pos, neg, user_table, item_table,
           W1r, b1r, W1rb, b1rb, W2r, b2r, W2rb, b2rb):
    src = edge_index[0]
    dst = edge_index[1]
    zf = jnp.zeros((NPAD, D), jnp.float32)
    ones = jnp.ones((K, D), jnp.float32)

    deg = _sc_deg(src, dst, zf, ones).reshape(2, NPAD, D)
    acc1 = _sc_agg(user_table, item_table, src, dst, zf).reshape(2, NPAD, D)

    W1s = jnp.stack([W1r, W1rb])
    b1s = jnp.stack([b1r, b1rb])[:, None, :]
    h1 = _tc_layer(acc1, deg, W1s, b1s, leaky=True)

    acc2 = _sc_agg(h1[1], h1[0], src, dst, zf).reshape(2, NPAD, D)
    W2s = jnp.stack([W2r, W2rb])
    b2s = jnp.stack([b2r, b2rb])[:, None, :]
    h2 = _tc_layer(acc2, deg, W2s, b2s, leaky=False)

    U, Pi, Ni = _sc_gather(h2[1], h2[0], uid,
                           pos.reshape(B), neg.reshape(B * NNEG))
    L = _tc_dots(U, Pi, Ni.reshape(B, NNEG * D))

    pos_logits = L[:, 0:1].reshape(B, 1, 1)
    neg_logits = L[:, 1:1 + NNEG].reshape(B, 1, NNEG)
    ue2 = jnp.broadcast_to(U[:, None, None, :], (B, 1, NNEG, D))
    pos_item_emb = Pi.reshape(B, 1, D)
    neg_item_emb = Ni.reshape(B, 1, NNEG, D)
    return pos_logits, neg_logits, ue2, pos_item_emb, neg_item_emb


# speedup vs baseline: 5.1344x; 1.6968x over previous
import functools

import jax
import jax.numpy as jnp
from jax import lax
from jax.experimental import pallas as pl
from jax.experimental.pallas import tpu as pltpu, tpu_sc as plsc

N = 10000
NPAD = 10240
D = 128
E = 320000
EPT = E // 16
K = 80
ITERS = EPT // K
CH = 10
NST = ITERS // CH
NBUF = 2
RPT = NPAD // 16
B = 4096
NNEG = 4

_MESH = plsc.VectorSubcoreMesh(core_axis_name="c", subcore_axis_name="s")


def _sc_agg(feat0, feat1, src3, dst3, zf):
    @functools.partial(
        pl.kernel,
        out_type=jax.ShapeDtypeStruct((2 * NPAD, D), jnp.float32),
        mesh=_MESH,
        scratch_types=[pltpu.VMEM((CH, K), jnp.int32),
                       pltpu.VMEM((CH, K), jnp.int32),
                       pltpu.VMEM((NBUF, K, D), jnp.float32),
                       pltpu.VMEM_SHARED((NPAD, D), jnp.float32)]
                      + [pltpu.SemaphoreType.DMA] * (2 * NBUF))
    def k(feat0_h, feat1_h, src4_h, dst4_h, zf_h, acc_o,
          idxi2, idxo2, rows, acc_sh, *sems):
        semg, semsc = sems[:NBUF], sems[NBUF:]
        c = lax.axis_index("c")
        s = lax.axis_index("s")
        r0 = s * RPT
        pltpu.sync_copy(zf_h.at[pl.ds(r0, RPT)], acc_sh.at[pl.ds(r0, RPT)])
        plsc.subcore_barrier()

        def pipeline(feat_h, in4_h, out4_h):
            @pl.loop(0, NST)
            def _stage(t):
                pltpu.sync_copy(in4_h.at[s, t], idxi2)
                pltpu.sync_copy(out4_h.at[s, t], idxo2)
                for b in range(NBUF):
                    pltpu.async_copy(feat_h.at[idxi2.at[b]],
                                     rows.at[b], semg[b])

                @pl.loop(0, CH, step=NBUF)
                def _round(i0):
                    for b in range(NBUF):
                        i = i0 + b
                        pltpu.make_async_copy(feat_h.at[idxi2.at[i]],
                                              rows.at[b], semg[b]).wait()
                        pltpu.async_copy(rows.at[b], acc_sh.at[idxo2.at[i]],
                                         semsc[b], add=True)
                    for b in range(NBUF):
                        i2 = i0 + NBUF + b
                        pltpu.make_async_copy(rows.at[b],
                                              acc_sh.at[idxo2.at[0]],
                                              semsc[b]).wait()

                        @pl.when(i2 < CH)
                        def _():
                            pltpu.async_copy(feat_h.at[idxi2.at[i2]],
                                             rows.at[b], semg[b])

        @pl.when(c == 0)
        def _():
            pipeline(feat0_h, src4_h, dst4_h)

        @pl.when(c == 1)
        def _():
            pipeline(feat1_h, dst4_h, src4_h)

        plsc.subcore_barrier()
        pltpu.sync_copy(acc_sh.at[pl.ds(r0, RPT)],
                        acc_o.at[pl.ds(c * NPAD + r0, RPT)])

    return k(feat0, feat1, src3, dst3, zf)


def _sc_deg(src3, dst3, zf, ones):
    @functools.partial(
        pl.kernel,
        out_type=jax.ShapeDtypeStruct((2 * NPAD, D), jnp.float32),
        mesh=_MESH,
        scratch_types=[pltpu.VMEM((CH, K), jnp.int32),
                       pltpu.VMEM((K, D), jnp.float32),
                       pltpu.VMEM_SHARED((NPAD, D), jnp.float32)]
                      + [pltpu.SemaphoreType.DMA] * NBUF)
    def k(src4_h, dst4_h, zf_h, ones_h, deg_o, idxo2, ones_v, deg_sh, *sems):
        c = lax.axis_index("c")
        s = lax.axis_index("s")
        r0 = s * RPT
        pltpu.sync_copy(zf_h.at[pl.ds(r0, RPT)], deg_sh.at[pl.ds(r0, RPT)])
        pltpu.sync_copy(ones_h, ones_v)
        plsc.subcore_barrier()

        def loop(out4_h):
            @pl.loop(0, NST)
            def _stage(t):
                pltpu.sync_copy(out4_h.at[s, t], idxo2)

                @pl.loop(0, CH, step=NBUF)
                def _round(i0):
                    for b in range(NBUF):
                        pltpu.async_copy(ones_v, deg_sh.at[idxo2.at[i0 + b]],
                                         sems[b], add=True)
                    for b in range(NBUF):
                        pltpu.make_async_copy(ones_v, deg_sh.at[idxo2.at[0]],
                                             sems[b]).wait()

        @pl.when(c == 0)
        def _():
            loop(dst4_h)

        @pl.when(c == 1)
        def _():
            loop(src4_h)

        plsc.subcore_barrier()
        pltpu.sync_copy(deg_sh.at[pl.ds(r0, RPT)],
                        deg_o.at[pl.ds(c * NPAD + r0, RPT)])

    return k(src3, dst3, zf, ones)


def _sc_gather(hu, hi, uid, posf, negf):
    @functools.partial(
        pl.kernel,
        out_type=[jax.ShapeDtypeStruct((B, D), jnp.float32),
                  jax.ShapeDtypeStruct((B, D), jnp.float32),
                  jax.ShapeDtypeStruct((B * NNEG, D), jnp.float32)],
        mesh=_MESH,
        scratch_types=[pltpu.VMEM((128,), jnp.int32),
                       pltpu.VMEM((128, D), jnp.float32),
                       pltpu.SemaphoreType.DMA])
    def k(hu_h, hi_h, uid_h, pos_h, neg_h, u_o, p_o, n_o, idx, rows, sem):
        c = lax.axis_index("c")
        s = lax.axis_index("s")
        wid = s * 2 + c

        def gath(tbl_h, ih, oh, base):
            pltpu.sync_copy(ih.at[pl.ds(base, 128)], idx)
            pltpu.async_copy(tbl_h.at[idx], rows, sem).wait()
            pltpu.sync_copy(rows, oh.at[pl.ds(base, 128)])

        gath(hu_h, uid_h, u_o, wid * 128)
        gath(hi_h, pos_h, p_o, wid * 128)
        for j in range(NNEG):
            gath(hi_h, neg_h, n_o, wid * 512 + j * 128)

    return k(hu, hi, uid, posf, negf)


def _tc_layer(acc, deg, Ws, bs, leaky):
    RB = 640

    def body(a_r, d_r, w_r, b_r, o_r):
        x = a_r[0]
        dcol = d_r[0][:, 0:1]
        m = x / jnp.maximum(dcol, 1.0)
        y = jnp.dot(m, w_r[0], preferred_element_type=jnp.float32)
        y = y + jnp.where(dcol > 0, b_r[0], 0.0)
        if leaky:
            y = jnp.where(y >= 0, y, 0.01 * y)
        o_r[0] = y

    return pl.pallas_call(
        body,
        grid=(2, NPAD // RB),
        in_specs=[pl.BlockSpec((1, RB, D), lambda d, i: (d, i, 0)),
                  pl.BlockSpec((1, RB, D), lambda d, i: (d, i, 0)),
                  pl.BlockSpec((1, D, D), lambda d, i: (d, 0, 0)),
                  pl.BlockSpec((1, 1, D), lambda d, i: (d, 0, 0))],
        out_specs=pl.BlockSpec((1, RB, D), lambda d, i: (d, i, 0)),
        out_shape=jax.ShapeDtypeStruct((2, NPAD, D), jnp.float32),
    )(acc, deg, Ws, bs)


def _tc_dots(U, Pi, N4):
    RB = 512

    def body(u_r, p_r, n_r, o_r):
        u = u_r[...]
        cols = [jnp.sum(u * p_r[...], axis=-1, keepdims=True)]
        for kk in range(NNEG):
            cols.append(jnp.sum(u * n_r[:, kk * D:(kk + 1) * D],
                                axis=-1, keepdims=True))
        cols.append(jnp.zeros((RB, 3), jnp.float32))
        o_r[...] = jnp.concatenate(cols, axis=-1)

    return pl.pallas_call(
        body,
        grid=(B // RB,),
        in_specs=[pl.BlockSpec((RB, D), lambda i: (i, 0)),
                  pl.BlockSpec((RB, D), lambda i: (i, 0)),
                  pl.BlockSpec((RB, NNEG * D), lambda i: (i, 0))],
        out_specs=pl.BlockSpec((RB, 8), lambda i: (i, 0)),
        out_shape=jax.ShapeDtypeStruct((B, 8), jnp.float32),
    )(U, Pi, N4)


def kernel(edge_index, uid, pos, neg, user_table, item_table,
           W1r, b1r, W1rb, b1rb, W2r, b2r, W2rb, b2rb):
    src3 = edge_index[0].reshape(16, NST, CH, K)
    dst3 = edge_index[1].reshape(16, NST, CH, K)
    zf = jnp.zeros((NPAD, D), jnp.float32)
    ones = jnp.ones((K, D), jnp.float32)

    deg = _sc_deg(src3, dst3, zf, ones).reshape(2, NPAD, D)
    acc1 = _sc_agg(user_table, item_table, src3, dst3, zf).reshape(2, NPAD, D)

    W1s = jnp.stack([W1r, W1rb])
    b1s = jnp.stack([b1r, b1rb])[:, None, :]
    h1 = _tc_layer(acc1, deg, W1s, b1s, leaky=True)

    acc2 = _sc_agg(h1[1], h1[0], src3, dst3, zf).reshape(2, NPAD, D)
    W2s = jnp.stack([W2r, W2rb])
    b2s = jnp.stack([b2r, b2rb])[:, None, :]
    h2 = _tc_layer(acc2, deg, W2s, b2s, leaky=False)

    U, Pi, Ni = _sc_gather(h2[1], h2[0], uid,
                           pos.reshape(B), neg.reshape(B * NNEG))
    L = _tc_dots(U, Pi, Ni.reshape(B, NNEG * D))

    pos_logits = L[:, 0:1].reshape(B, 1, 1)
    neg_logits = L[:, 1:1 + NNEG].reshape(B, 1, NNEG)
    ue2 = jnp.broadcast_to(U[:, None, None, :], (B, 1, NNEG, D))
    pos_item_emb = Pi.reshape(B, 1, D)
    neg_item_emb = Ni.reshape(B, 1, NNEG, D)
    return pos_logits, neg_logits, ue2, pos_item_emb, neg_item_emb


# speedup vs baseline: 5.7589x; 1.1216x over previous
import functools

import jax
import jax.numpy as jnp
from jax import lax
from jax.experimental import pallas as pl
from jax.experimental.pallas import tpu as pltpu, tpu_sc as plsc

N = 10000
NPAD = 10240
D = 128
E = 320000
EPT = E // 16
K = 40
ITERS = EPT // K
CH = 20
NST = ITERS // CH
NBUF = 4
RPT = NPAD // 16
B = 4096
NNEG = 4

_MESH = plsc.VectorSubcoreMesh(core_axis_name="c", subcore_axis_name="s")


def _sc_agg(feat0, feat1, src3, dst3, zf):
    @functools.partial(
        pl.kernel,
        out_type=jax.ShapeDtypeStruct((2 * NPAD, D), jnp.float32),
        mesh=_MESH,
        scratch_types=[pltpu.VMEM((CH, K), jnp.int32),
                       pltpu.VMEM((CH, K), jnp.int32),
                       pltpu.VMEM((NBUF, K, D), jnp.float32),
                       pltpu.VMEM_SHARED((NPAD, D), jnp.float32)]
                      + [pltpu.SemaphoreType.DMA] * (2 * NBUF))
    def k(feat0_h, feat1_h, src4_h, dst4_h, zf_h, acc_o,
          idxi2, idxo2, rows, acc_sh, *sems):
        semg, semsc = sems[:NBUF], sems[NBUF:]
        c = lax.axis_index("c")
        s = lax.axis_index("s")
        r0 = s * RPT
        pltpu.sync_copy(zf_h.at[pl.ds(r0, RPT)], acc_sh.at[pl.ds(r0, RPT)])
        plsc.subcore_barrier()

        def pipeline(feat_h, in4_h, out4_h):
            @pl.loop(0, NST)
            def _stage(t):
                pltpu.sync_copy(in4_h.at[s, t], idxi2)
                pltpu.sync_copy(out4_h.at[s, t], idxo2)
                for b in range(NBUF):
                    pltpu.async_copy(feat_h.at[idxi2.at[b]],
                                     rows.at[b], semg[b])

                @pl.loop(0, CH, step=NBUF)
                def _round(i0):
                    for b in range(NBUF):
                        i = i0 + b
                        pltpu.make_async_copy(feat_h.at[idxi2.at[i]],
                                              rows.at[b], semg[b]).wait()
                        pltpu.async_copy(rows.at[b], acc_sh.at[idxo2.at[i]],
                                         semsc[b], add=True)
                    for b in range(NBUF):
                        i2 = i0 + NBUF + b
                        pltpu.make_async_copy(rows.at[b],
                                              acc_sh.at[idxo2.at[0]],
                                              semsc[b]).wait()

                        @pl.when(i2 < CH)
                        def _():
                            pltpu.async_copy(feat_h.at[idxi2.at[i2]],
                                             rows.at[b], semg[b])

        @pl.when(c == 0)
        def _():
            pipeline(feat0_h, src4_h, dst4_h)

        @pl.when(c == 1)
        def _():
            pipeline(feat1_h, dst4_h, src4_h)

        plsc.subcore_barrier()
        pltpu.sync_copy(acc_sh.at[pl.ds(r0, RPT)],
                        acc_o.at[pl.ds(c * NPAD + r0, RPT)])

    return k(feat0, feat1, src3, dst3, zf)


DW = D


def _sc_deg(src3, dst3, zd, ones):
    @functools.partial(
        pl.kernel,
        out_type=jax.ShapeDtypeStruct((2 * NPAD, DW), jnp.float32),
        mesh=_MESH,
        scratch_types=[pltpu.VMEM((CH, K), jnp.int32),
                       pltpu.VMEM((K, DW), jnp.float32),
                       pltpu.VMEM_SHARED((NPAD, DW), jnp.float32)]
                      + [pltpu.SemaphoreType.DMA] * NBUF)
    def k(src4_h, dst4_h, zd_h, ones_h, deg_o, idxo2, ones_v, deg_sh, *sems):
        c = lax.axis_index("c")
        s = lax.axis_index("s")
        r0 = s * RPT
        pltpu.sync_copy(zd_h.at[pl.ds(r0, RPT)], deg_sh.at[pl.ds(r0, RPT)])
        pltpu.sync_copy(ones_h, ones_v)
        plsc.subcore_barrier()

        def loop(out4_h):
            @pl.loop(0, NST)
            def _stage(t):
                pltpu.sync_copy(out4_h.at[s, t], idxo2)

                @pl.loop(0, CH, step=NBUF)
                def _round(i0):
                    for b in range(NBUF):
                        pltpu.async_copy(ones_v, deg_sh.at[idxo2.at[i0 + b]],
                                         sems[b], add=True)
                    for b in range(NBUF):
                        pltpu.make_async_copy(ones_v, deg_sh.at[idxo2.at[0]],
                                             sems[b]).wait()

        @pl.when(c == 0)
        def _():
            loop(dst4_h)

        @pl.when(c == 1)
        def _():
            loop(src4_h)

        plsc.subcore_barrier()
        pltpu.sync_copy(deg_sh.at[pl.ds(r0, RPT)],
                        deg_o.at[pl.ds(c * NPAD + r0, RPT)])

    return k(src3, dst3, zd, ones)


def _sc_gather(hu, hi, uid, posf, negf):
    @functools.partial(
        pl.kernel,
        out_type=[jax.ShapeDtypeStruct((B, D), jnp.float32),
                  jax.ShapeDtypeStruct((B, D), jnp.float32),
                  jax.ShapeDtypeStruct((B * NNEG, D), jnp.float32)],
        mesh=_MESH,
        scratch_types=[pltpu.VMEM((128,), jnp.int32),
                       pltpu.VMEM((128, D), jnp.float32),
                       pltpu.SemaphoreType.DMA])
    def k(hu_h, hi_h, uid_h, pos_h, neg_h, u_o, p_o, n_o, idx, rows, sem):
        c = lax.axis_index("c")
        s = lax.axis_index("s")
        wid = s * 2 + c

        def gath(tbl_h, ih, oh, base):
            pltpu.sync_copy(ih.at[pl.ds(base, 128)], idx)
            pltpu.async_copy(tbl_h.at[idx], rows, sem).wait()
            pltpu.sync_copy(rows, oh.at[pl.ds(base, 128)])

        gath(hu_h, uid_h, u_o, wid * 128)
        gath(hi_h, pos_h, p_o, wid * 128)
        for j in range(NNEG):
            gath(hi_h, neg_h, n_o, wid * 512 + j * 128)

    return k(hu, hi, uid, posf, negf)


def _tc_layer(acc, deg, Ws, bs, leaky):
    RB = 640

    def body(a_r, d_r, w_r, b_r, o_r):
        x = a_r[0]
        dcol = d_r[0][:, 0:1]
        m = x / jnp.maximum(dcol, 1.0)
        y = jnp.dot(m, w_r[0], preferred_element_type=jnp.float32)
        y = y + jnp.where(dcol > 0, b_r[0], 0.0)
        if leaky:
            y = jnp.where(y >= 0, y, 0.01 * y)
        o_r[0] = y

    return pl.pallas_call(
        body,
        grid=(2, NPAD // RB),
        in_specs=[pl.BlockSpec((1, RB, D), lambda d, i: (d, i, 0)),
                  pl.BlockSpec((1, RB, DW), lambda d, i: (d, i, 0)),
                  pl.BlockSpec((1, D, D), lambda d, i: (d, 0, 0)),
                  pl.BlockSpec((1, 1, D), lambda d, i: (d, 0, 0))],
        out_specs=pl.BlockSpec((1, RB, D), lambda d, i: (d, i, 0)),
        out_shape=jax.ShapeDtypeStruct((2, NPAD, D), jnp.float32),
    )(acc, deg, Ws, bs)


def _tc_dots(U, Pi, N4):
    RB = 512

    def body(u_r, p_r, n_r, o_r):
        u = u_r[...]
        cols = [jnp.sum(u * p_r[...], axis=-1, keepdims=True)]
        for kk in range(NNEG):
            cols.append(jnp.sum(u * n_r[:, kk * D:(kk + 1) * D],
                                axis=-1, keepdims=True))
        cols.append(jnp.zeros((RB, 3), jnp.float32))
        o_r[...] = jnp.concatenate(cols, axis=-1)

    return pl.pallas_call(
        body,
        grid=(B // RB,),
        in_specs=[pl.BlockSpec((RB, D), lambda i: (i, 0)),
                  pl.BlockSpec((RB, D), lambda i: (i, 0)),
                  pl.BlockSpec((RB, NNEG * D), lambda i: (i, 0))],
        out_specs=pl.BlockSpec((RB, 8), lambda i: (i, 0)),
        out_shape=jax.ShapeDtypeStruct((B, 8), jnp.float32),
    )(U, Pi, N4)


def kernel(edge_index, uid, pos, neg, user_table, item_table,
           W1r, b1r, W1rb, b1rb, W2r, b2r, W2rb, b2rb):
    src3 = edge_index[0].reshape(16, NST, CH, K)
    dst3 = edge_index[1].reshape(16, NST, CH, K)
    zf = jnp.zeros((NPAD, D), jnp.float32)
    zd = jnp.zeros((NPAD, DW), jnp.float32)
    ones = jnp.ones((K, DW), jnp.float32)

    deg = _sc_deg(src3, dst3, zd, ones).reshape(2, NPAD, DW)
    acc1 = _sc_agg(user_table, item_table, src3, dst3, zf).reshape(2, NPAD, D)

    W1s = jnp.stack([W1r, W1rb])
    b1s = jnp.stack([b1r, b1rb])[:, None, :]
    h1 = _tc_layer(acc1, deg, W1s, b1s, leaky=True)

    acc2 = _sc_agg(h1[1], h1[0], src3, dst3, zf).reshape(2, NPAD, D)
    W2s = jnp.stack([W2r, W2rb])
    b2s = jnp.stack([b2r, b2rb])[:, None, :]
    h2 = _tc_layer(acc2, deg, W2s, b2s, leaky=False)

    U, Pi, Ni = _sc_gather(h2[1], h2[0], uid,
                           pos.reshape(B), neg.reshape(B * NNEG))
    L = _tc_dots(U, Pi, Ni.reshape(B, NNEG * D))

    pos_logits = L[:, 0:1].reshape(B, 1, 1)
    neg_logits = L[:, 1:1 + NNEG].reshape(B, 1, NNEG)
    ue2 = jnp.broadcast_to(U[:, None, None, :], (B, 1, NNEG, D))
    pos_item_emb = Pi.reshape(B, 1, D)
    neg_item_emb = Ni.reshape(B, 1, NNEG, D)
    return pos_logits, neg_logits, ue2, pos_item_emb, neg_item_emb


# speedup vs baseline: 5.8650x; 1.0184x over previous
import functools

import jax
import jax.numpy as jnp
from jax import lax
from jax.experimental import pallas as pl
from jax.experimental.pallas import tpu as pltpu, tpu_sc as plsc

N = 10000
NPAD = 10240
D = 128
E = 320000
EPT = E // 16
K = 40
ITERS = EPT // K
CH = 20
NST = ITERS // CH
NBUF = 5
RPT = NPAD // 16
B = 4096
NNEG = 4

_MESH = plsc.VectorSubcoreMesh(core_axis_name="c", subcore_axis_name="s")


def _sc_agg(feat0, feat1, src3, dst3, zf):
    @functools.partial(
        pl.kernel,
        out_type=jax.ShapeDtypeStruct((2 * NPAD, D), jnp.float32),
        mesh=_MESH,
        scratch_types=[pltpu.VMEM((CH, K), jnp.int32),
                       pltpu.VMEM((CH, K), jnp.int32),
                       pltpu.VMEM((NBUF, K, D), jnp.float32),
                       pltpu.VMEM_SHARED((NPAD, D), jnp.float32)]
                      + [pltpu.SemaphoreType.DMA] * (2 * NBUF))
    def k(feat0_h, feat1_h, src4_h, dst4_h, zf_h, acc_o,
          idxi2, idxo2, rows, acc_sh, *sems):
        semg, semsc = sems[:NBUF], sems[NBUF:]
        c = lax.axis_index("c")
        s = lax.axis_index("s")
        r0 = s * RPT
        pltpu.sync_copy(zf_h.at[pl.ds(r0, RPT)], acc_sh.at[pl.ds(r0, RPT)])
        plsc.subcore_barrier()

        def pipeline(feat_h, in4_h, out4_h):
            @pl.loop(0, NST)
            def _stage(t):
                pltpu.sync_copy(in4_h.at[s, t], idxi2)
                pltpu.sync_copy(out4_h.at[s, t], idxo2)
                for b in range(NBUF):
                    pltpu.async_copy(feat_h.at[idxi2.at[b]],
                                     rows.at[b], semg[b])

                @pl.loop(0, CH, step=NBUF)
                def _round(i0):
                    for b in range(NBUF):
                        i = i0 + b
                        pltpu.make_async_copy(feat_h.at[idxi2.at[i]],
                                              rows.at[b], semg[b]).wait()
                        pltpu.async_copy(rows.at[b], acc_sh.at[idxo2.at[i]],
                                         semsc[b], add=True)
                    for b in range(NBUF):
                        i2 = i0 + NBUF + b
                        pltpu.make_async_copy(rows.at[b],
                                              acc_sh.at[idxo2.at[0]],
                                              semsc[b]).wait()

                        @pl.when(i2 < CH)
                        def _():
                            pltpu.async_copy(feat_h.at[idxi2.at[i2]],
                                             rows.at[b], semg[b])

        @pl.when(c == 0)
        def _():
            pipeline(feat0_h, src4_h, dst4_h)

        @pl.when(c == 1)
        def _():
            pipeline(feat1_h, dst4_h, src4_h)

        plsc.subcore_barrier()
        pltpu.sync_copy(acc_sh.at[pl.ds(r0, RPT)],
                        acc_o.at[pl.ds(c * NPAD + r0, RPT)])

    return k(feat0, feat1, src3, dst3, zf)


DW = D


def _sc_deg(src3, dst3, zd, ones):
    @functools.partial(
        pl.kernel,
        out_type=jax.ShapeDtypeStruct((2 * NPAD, DW), jnp.float32),
        mesh=_MESH,
        scratch_types=[pltpu.VMEM((CH, K), jnp.int32),
                       pltpu.VMEM((K, DW), jnp.float32),
                       pltpu.VMEM_SHARED((NPAD, DW), jnp.float32)]
                      + [pltpu.SemaphoreType.DMA] * NBUF)
    def k(src4_h, dst4_h, zd_h, ones_h, deg_o, idxo2, ones_v, deg_sh, *sems):
        c = lax.axis_index("c")
        s = lax.axis_index("s")
        r0 = s * RPT
        pltpu.sync_copy(zd_h.at[pl.ds(r0, RPT)], deg_sh.at[pl.ds(r0, RPT)])
        pltpu.sync_copy(ones_h, ones_v)
        plsc.subcore_barrier()

        def loop(out4_h):
            @pl.loop(0, NST)
            def _stage(t):
                pltpu.sync_copy(out4_h.at[s, t], idxo2)

                @pl.loop(0, CH, step=NBUF)
                def _round(i0):
                    for b in range(NBUF):
                        pltpu.async_copy(ones_v, deg_sh.at[idxo2.at[i0 + b]],
                                         sems[b], add=True)
                    for b in range(NBUF):
                        pltpu.make_async_copy(ones_v, deg_sh.at[idxo2.at[0]],
                                             sems[b]).wait()

        @pl.when(c == 0)
        def _():
            loop(dst4_h)

        @pl.when(c == 1)
        def _():
            loop(src4_h)

        plsc.subcore_barrier()
        pltpu.sync_copy(deg_sh.at[pl.ds(r0, RPT)],
                        deg_o.at[pl.ds(c * NPAD + r0, RPT)])

    return k(src3, dst3, zd, ones)


def _sc_gather(hu, hi, uid, posf, negf):
    @functools.partial(
        pl.kernel,
        out_type=[jax.ShapeDtypeStruct((B, D), jnp.float32),
                  jax.ShapeDtypeStruct((B, D), jnp.float32),
                  jax.ShapeDtypeStruct((B * NNEG, D), jnp.float32)],
        mesh=_MESH,
        scratch_types=[pltpu.VMEM((128,), jnp.int32),
                       pltpu.VMEM((128, D), jnp.float32),
                       pltpu.SemaphoreType.DMA])
    def k(hu_h, hi_h, uid_h, pos_h, neg_h, u_o, p_o, n_o, idx, rows, sem):
        c = lax.axis_index("c")
        s = lax.axis_index("s")
        wid = s * 2 + c

        def gath(tbl_h, ih, oh, base):
            pltpu.sync_copy(ih.at[pl.ds(base, 128)], idx)
            pltpu.async_copy(tbl_h.at[idx], rows, sem).wait()
            pltpu.sync_copy(rows, oh.at[pl.ds(base, 128)])

        gath(hu_h, uid_h, u_o, wid * 128)
        gath(hi_h, pos_h, p_o, wid * 128)
        for j in range(NNEG):
            gath(hi_h, neg_h, n_o, wid * 512 + j * 128)

    return k(hu, hi, uid, posf, negf)


def _tc_layer(acc, deg, Ws, bs, leaky):
    RB = 640

    def body(a_r, d_r, w_r, b_r, o_r):
        x = a_r[0]
        dcol = d_r[0][:, 0:1]
        m = x / jnp.maximum(dcol, 1.0)
        y = jnp.dot(m, w_r[0], preferred_element_type=jnp.float32)
        y = y + jnp.where(dcol > 0, b_r[0], 0.0)
        if leaky:
            y = jnp.where(y >= 0, y, 0.01 * y)
        o_r[0] = y

    return pl.pallas_call(
        body,
        grid=(2, NPAD // RB),
        in_specs=[pl.BlockSpec((1, RB, D), lambda d, i: (d, i, 0)),
                  pl.BlockSpec((1, RB, DW), lambda d, i: (d, i, 0)),
                  pl.BlockSpec((1, D, D), lambda d, i: (d, 0, 0)),
                  pl.BlockSpec((1, 1, D), lambda d, i: (d, 0, 0))],
        out_specs=pl.BlockSpec((1, RB, D), lambda d, i: (d, i, 0)),
        out_shape=jax.ShapeDtypeStruct((2, NPAD, D), jnp.float32),
    )(acc, deg, Ws, bs)


def _tc_dots(U, Pi, N4):
    RB = 512

    def body(u_r, p_r, n_r, o_r):
        u = u_r[...]
        cols = [jnp.sum(u * p_r[...], axis=-1, keepdims=True)]
        for kk in range(NNEG):
            cols.append(jnp.sum(u * n_r[:, kk * D:(kk + 1) * D],
                                axis=-1, keepdims=True))
        cols.append(jnp.zeros((RB, 3), jnp.float32))
        o_r[...] = jnp.concatenate(cols, axis=-1)

    return pl.pallas_call(
        body,
        grid=(B // RB,),
        in_specs=[pl.BlockSpec((RB, D), lambda i: (i, 0)),
                  pl.BlockSpec((RB, D), lambda i: (i, 0)),
                  pl.BlockSpec((RB, NNEG * D), lambda i: (i, 0))],
        out_specs=pl.BlockSpec((RB, 8), lambda i: (i, 0)),
        out_shape=jax.ShapeDtypeStruct((B, 8), jnp.float32),
    )(U, Pi, N4)


def kernel(edge_index, uid, pos, neg, user_table, item_table,
           W1r, b1r, W1rb, b1rb, W2r, b2r, W2rb, b2rb):
    src3 = edge_index[0].reshape(16, NST, CH, K)
    dst3 = edge_index[1].reshape(16, NST, CH, K)
    zf = jnp.zeros((NPAD, D), jnp.float32)
    zd = jnp.zeros((NPAD, DW), jnp.float32)
    ones = jnp.ones((K, DW), jnp.float32)

    deg = _sc_deg(src3, dst3, zd, ones).reshape(2, NPAD, DW)
    acc1 = _sc_agg(user_table, item_table, src3, dst3, zf).reshape(2, NPAD, D)

    W1s = jnp.stack([W1r, W1rb])
    b1s = jnp.stack([b1r, b1rb])[:, None, :]
    h1 = _tc_layer(acc1, deg, W1s, b1s, leaky=True)

    acc2 = _sc_agg(h1[1], h1[0], src3, dst3, zf).reshape(2, NPAD, D)
    W2s = jnp.stack([W2r, W2rb])
    b2s = jnp.stack([b2r, b2rb])[:, None, :]
    h2 = _tc_layer(acc2, deg, W2s, b2s, leaky=False)

    U, Pi, Ni = _sc_gather(h2[1], h2[0], uid,
                           pos.reshape(B), neg.reshape(B * NNEG))
    L = _tc_dots(U, Pi, Ni.reshape(B, NNEG * D))

    pos_logits = L[:, 0:1].reshape(B, 1, 1)
    neg_logits = L[:, 1:1 + NNEG].reshape(B, 1, NNEG)
    ue2 = jnp.broadcast_to(U[:, None, None, :], (B, 1, NNEG, D))
    pos_item_emb = Pi.reshape(B, 1, D)
    neg_item_emb = Ni.reshape(B, 1, NNEG, D)
    return pos_logits, neg_logits, ue2, pos_item_emb, neg_item_emb
